# baseline (speedup 1.0000x reference)
import functools

import jax
import jax.numpy as jnp
from jax import lax
from jax.experimental import pallas as pl
from jax.experimental.pallas import tpu as pltpu
from jax.experimental.pallas import tpu_sc as plsc

EPS = 1e-5
SUB = 128
CHUNK = 512
GROUP = 16


def _rsqrt(v):
    y = plsc.bitcast(
        jnp.int32(0x5F3759DF) - lax.shift_right_logical(plsc.bitcast(v, jnp.int32), 1),
        jnp.float32,
    )
    for _ in range(3):
        y = y * (1.5 - 0.5 * v * y * y)
    return y


def _make_kernel(n_idx, num_emb, d):
    try:
        info = plsc.get_sparse_core_info()
        num_cores, num_subcores = info.num_cores, info.num_subcores
    except ValueError:
        num_cores, num_subcores = 2, 16
    nw = num_cores * num_subcores
    per_w = n_idx // nw
    chunks = per_w // CHUNK
    n_sub = CHUNK // SUB
    groups = CHUNK // GROUP
    assert per_w % (2 * CHUNK) == 0 and n_idx % nw == 0 and CHUNK == 4 * SUB

    mesh = plsc.VectorSubcoreMesh(
        core_axis_name="c", subcore_axis_name="s",
        num_cores=num_cores, num_subcores=num_subcores,
    )

    @functools.partial(
        pl.kernel,
        mesh=mesh,
        out_type=jax.ShapeDtypeStruct((n_idx, d), jnp.float32),
        compiler_params=pltpu.CompilerParams(
            use_tc_tiling_on_sc=False, needs_layout_passes=False),
        scratch_types=[
            pltpu.VMEM((2 * n_sub, SUB), jnp.int32),
            pltpu.VMEM((CHUNK, d), jnp.float32),
            pltpu.VMEM((d,), jnp.float32),
            pltpu.VMEM((d,), jnp.float32),
            pltpu.SemaphoreType.DMA,
        ],
    )
    def kern(x_hbm, table_hbm, gamma_hbm, beta_hbm, out_hbm,
             idx_v, rows_v, gamma_v, beta_v, sem):
        wid = lax.axis_index("s") * num_cores + lax.axis_index("c")
        pltpu.sync_copy(gamma_hbm, gamma_v)
        pltpu.sync_copy(beta_hbm, beta_v)
        lane = lax.iota(jnp.int32, 16)
        inv_d = jnp.float32(1.0 / d)
        gq = [gamma_v[pl.ds(q * 16, 16)] for q in range(d // 16)]
        bq = [beta_v[pl.ds(q * 16, 16)] for q in range(d // 16)]

        def duo_body(k2, carry):
            row0 = wid * (per_w // SUB) + k2 * 2 * n_sub
            pltpu.sync_copy(x_hbm.at[pl.ds(row0, 2 * n_sub)], idx_v)
            for h in range(2):
                _do_chunk(k2 * 2 + h, h)
            return carry

        def _do_chunk(k, h):
            base = wid * per_w + k * CHUNK
            copies = [
                pltpu.async_copy(
                    table_hbm.at[idx_v.at[h * n_sub + j]],
                    rows_v.at[pl.ds(j * SUB, SUB)],
                    sem,
                )
                for j in range(n_sub)
            ]
            for c in copies:
                c.wait()

            def group_body(g, carry2):
                row = lane + g * GROUP
                accs = [jnp.zeros((16,), jnp.float32) for _ in range(2)]
                accs2 = [jnp.zeros((16,), jnp.float32) for _ in range(2)]
                for dd in range(d):
                    cvec = (lane + dd) & (d - 1)
                    col = plsc.load_gather(rows_v, [row, cvec])
                    accs[dd % 2] = accs[dd % 2] + col
                    accs2[dd % 2] = accs2[dd % 2] + col * col
                mean = (accs[0] + accs[1]) * inv_d
                var = (accs2[0] + accs2[1]) * inv_d - mean * mean
                rstd = _rsqrt(jnp.maximum(var, 0.0) + EPS)
                for i in range(GROUP):
                    r = g * GROUP + i
                    m_i = mean[i]
                    rs_i = rstd[i]
                    for q in range(d // 16):
                        v = rows_v[r, pl.ds(q * 16, 16)]
                        rows_v[r, pl.ds(q * 16, 16)] = (
                            (v - m_i) * rs_i * gq[q] + bq[q]
                        )
                return carry2

            lax.fori_loop(0, groups, group_body, 0)
            pltpu.sync_copy(rows_v, out_hbm.at[pl.ds(base, CHUNK)])

        lax.fori_loop(0, chunks // 2, duo_body, 0)

    return kern


def kernel(x, table, gamma, beta):
    b, s = x.shape
    num_emb, d = table.shape
    n_idx = b * s
    kern = _make_kernel(n_idx, num_emb, d)
    x_flat = x.reshape(n_idx // SUB, SUB)
    out = kern(x_flat, table, gamma, beta)
    return out.reshape(b, s, d)


# speedup vs baseline: 1.0455x; 1.0455x over previous
import functools

import jax
import jax.numpy as jnp
from jax import lax
from jax.experimental import pallas as pl
from jax.experimental.pallas import tpu as pltpu
from jax.experimental.pallas import tpu_sc as plsc

EPS = 1e-5
SUB = 128
CHUNK = 512
GROUP = 16


def _rsqrt(v):
    y = plsc.bitcast(
        jnp.int32(0x5F3759DF) - lax.shift_right_logical(plsc.bitcast(v, jnp.int32), 1),
        jnp.float32,
    )
    for _ in range(3):
        y = y * (1.5 - 0.5 * v * y * y)
    return y


def _make_kernel(n_idx, num_emb, d):
    try:
        info = plsc.get_sparse_core_info()
        num_cores, num_subcores = info.num_cores, info.num_subcores
    except ValueError:
        num_cores, num_subcores = 2, 16
    nw = num_cores * num_subcores
    per_w = n_idx // nw
    chunks = per_w // CHUNK
    n_sub = CHUNK // SUB
    groups = CHUNK // GROUP
    assert per_w % (2 * CHUNK) == 0 and n_idx % nw == 0 and CHUNK == 4 * SUB

    mesh = plsc.VectorSubcoreMesh(
        core_axis_name="c", subcore_axis_name="s",
        num_cores=num_cores, num_subcores=num_subcores,
    )

    @functools.partial(
        pl.kernel,
        mesh=mesh,
        out_type=jax.ShapeDtypeStruct((n_idx, d), jnp.float32),
        compiler_params=pltpu.CompilerParams(
            use_tc_tiling_on_sc=False, needs_layout_passes=False),
        scratch_types=[
            pltpu.VMEM((2, n_sub, SUB), jnp.int32),
            pltpu.VMEM((CHUNK, d), jnp.float32),
            pltpu.VMEM((CHUNK, d), jnp.float32),
            pltpu.VMEM((d,), jnp.float32),
            pltpu.VMEM((d,), jnp.float32),
            pltpu.SemaphoreType.DMA,
            pltpu.SemaphoreType.DMA,
            pltpu.SemaphoreType.DMA,
            pltpu.SemaphoreType.DMA,
        ],
    )
    def kern(x_hbm, table_hbm, gamma_hbm, beta_hbm, out_hbm,
             idx_v, rows_a, rows_b, gamma_v, beta_v,
             gsem_a, gsem_b, wsem_a, wsem_b):
        wid = lax.axis_index("s") * num_cores + lax.axis_index("c")
        pltpu.sync_copy(gamma_hbm, gamma_v)
        pltpu.sync_copy(beta_hbm, beta_v)
        lane = lax.iota(jnp.int32, 16)
        inv_d = jnp.float32(1.0 / d)
        gq = [gamma_v[pl.ds(q * 16, 16)] for q in range(d // 16)]
        bq = [beta_v[pl.ds(q * 16, 16)] for q in range(d // 16)]
        c0 = wid * chunks

        def stage_and_fire(c, slot, rows_ref, gsem):
            pltpu.sync_copy(x_hbm.at[pl.ds(c * n_sub, n_sub)], idx_v.at[slot])
            for j in range(n_sub):
                pltpu.async_copy(
                    table_hbm.at[idx_v.at[slot, j]],
                    rows_ref.at[pl.ds(j * SUB, SUB)],
                    gsem,
                )

        def drain_gathers(rows_ref, gsem):
            pltpu.make_async_copy(
                table_hbm.at[pl.ds(0, CHUNK)], rows_ref, gsem).wait()

        def fire_write(c, rows_ref, wsem):
            pltpu.async_copy(rows_ref, out_hbm.at[pl.ds(c * CHUNK, CHUNK)], wsem)

        def drain_write(rows_ref, wsem):
            pltpu.make_async_copy(
                rows_ref, out_hbm.at[pl.ds(0, CHUNK)], wsem).wait()

        def compute(rows_v):
            def group_body(g, carry2):
                row = lane + g * GROUP
                accs = [jnp.zeros((16,), jnp.float32) for _ in range(2)]
                accs2 = [jnp.zeros((16,), jnp.float32) for _ in range(2)]
                for dd in range(d):
                    cvec = (lane + dd) & (d - 1)
                    col = plsc.load_gather(rows_v, [row, cvec])
                    accs[dd % 2] = accs[dd % 2] + col
                    accs2[dd % 2] = accs2[dd % 2] + col * col
                mean = (accs[0] + accs[1]) * inv_d
                var = (accs2[0] + accs2[1]) * inv_d - mean * mean
                rstd = _rsqrt(jnp.maximum(var, 0.0) + EPS)
                for i in range(GROUP):
                    r = g * GROUP + i
                    m_i = mean[i]
                    rs_i = rstd[i]
                    for q in range(d // 16):
                        v = rows_v[r, pl.ds(q * 16, 16)]
                        rows_v[r, pl.ds(q * 16, 16)] = (
                            (v - m_i) * rs_i * gq[q] + bq[q]
                        )
                return carry2

            lax.fori_loop(0, groups, group_body, 0)

        npairs = chunks // 2
        stage_and_fire(c0, 0, rows_a, gsem_a)

        def pair_body(t, carry):
            ca = c0 + 2 * t
            drain_gathers(rows_a, gsem_a)

            @pl.when(t > 0)
            def _():
                drain_write(rows_b, wsem_b)

            stage_and_fire(ca + 1, 1, rows_b, gsem_b)
            compute(rows_a)
            fire_write(ca, rows_a, wsem_a)
            drain_gathers(rows_b, gsem_b)

            @pl.when(t < npairs - 1)
            def _():
                drain_write(rows_a, wsem_a)
                stage_and_fire(ca + 2, 0, rows_a, gsem_a)

            compute(rows_b)
            fire_write(ca + 1, rows_b, wsem_b)
            return carry

        lax.fori_loop(0, npairs, pair_body, 0)
        drain_write(rows_a, wsem_a)
        drain_write(rows_b, wsem_b)

    return kern


def kernel(x, table, gamma, beta):
    b, s = x.shape
    num_emb, d = table.shape
    n_idx = b * s
    kern = _make_kernel(n_idx, num_emb, d)
    x_flat = x.reshape(n_idx // SUB, SUB)
    out = kern(x_flat, table, gamma, beta)
    return out.reshape(b, s, d)


# speedup vs baseline: 1.4790x; 1.4146x over previous
import functools

import jax
import jax.numpy as jnp
from jax import lax
from jax.experimental import pallas as pl
from jax.experimental.pallas import tpu as pltpu
from jax.experimental.pallas import tpu_sc as plsc

EPS = 1e-5
SUB = 128
CHUNK = 512
GROUP = 16


def _rsqrt(v):
    y = plsc.bitcast(
        jnp.int32(0x5F3759DF) - lax.shift_right_logical(plsc.bitcast(v, jnp.int32), 1),
        jnp.float32,
    )
    for _ in range(2):
        y = y * (1.5 - 0.5 * v * y * y)
    return y


def _make_kernel(n_idx, num_emb, d):
    try:
        info = plsc.get_sparse_core_info()
        num_cores, num_subcores = info.num_cores, info.num_subcores
    except ValueError:
        num_cores, num_subcores = 2, 16
    nw = num_cores * num_subcores
    per_w = n_idx // nw
    chunks = per_w // CHUNK
    n_sub = CHUNK // SUB
    groups = CHUNK // GROUP
    assert per_w % (2 * CHUNK) == 0 and n_idx % nw == 0 and CHUNK == 4 * SUB

    mesh = plsc.VectorSubcoreMesh(
        core_axis_name="c", subcore_axis_name="s",
        num_cores=num_cores, num_subcores=num_subcores,
    )

    @functools.partial(
        pl.kernel,
        mesh=mesh,
        out_type=jax.ShapeDtypeStruct((n_idx, d), jnp.float32),
        compiler_params=pltpu.CompilerParams(
            use_tc_tiling_on_sc=False, needs_layout_passes=False),
        scratch_types=[
            pltpu.VMEM((2, n_sub, SUB), jnp.int32),
            pltpu.VMEM((CHUNK, d), jnp.float32),
            pltpu.VMEM((CHUNK, d), jnp.float32),
            pltpu.VMEM((d,), jnp.float32),
            pltpu.VMEM((d,), jnp.float32),
            pltpu.SemaphoreType.DMA,
            pltpu.SemaphoreType.DMA,
            pltpu.SemaphoreType.DMA,
            pltpu.SemaphoreType.DMA,
        ],
    )
    def kern(x_hbm, table_hbm, gamma_hbm, beta_hbm, out_hbm,
             idx_v, rows_a, rows_b, gamma_v, beta_v,
             gsem_a, gsem_b, wsem_a, wsem_b):
        wid = lax.axis_index("s") * num_cores + lax.axis_index("c")
        pltpu.sync_copy(gamma_hbm, gamma_v)
        pltpu.sync_copy(beta_hbm, beta_v)
        inv_d = jnp.float32(1.0 / d)
        gq = [gamma_v[pl.ds(q * 16, 16)] for q in range(d // 16)]
        bq = [beta_v[pl.ds(q * 16, 16)] for q in range(d // 16)]
        c0 = wid * chunks

        def stage_and_fire(c, slot, rows_ref, gsem):
            pltpu.sync_copy(x_hbm.at[pl.ds(c * n_sub, n_sub)], idx_v.at[slot])
            for j in range(n_sub):
                pltpu.async_copy(
                    table_hbm.at[idx_v.at[slot, j]],
                    rows_ref.at[pl.ds(j * SUB, SUB)],
                    gsem,
                )

        def drain_gathers(rows_ref, gsem):
            pltpu.make_async_copy(
                table_hbm.at[pl.ds(0, CHUNK)], rows_ref, gsem).wait()

        def fire_write(c, rows_ref, wsem):
            pltpu.async_copy(rows_ref, out_hbm.at[pl.ds(c * CHUNK, CHUNK)], wsem)

        def drain_write(rows_ref, wsem):
            pltpu.make_async_copy(
                rows_ref, out_hbm.at[pl.ds(0, CHUNK)], wsem).wait()

        def compute(rows_v):
            nq = d // 16

            def one_row(r):
                v = [rows_v[r, pl.ds(q * 16, 16)] for q in range(nq)]
                s = (v[0] + v[1]) + (v[2] + v[3])
                sq = (v[0] * v[0] + v[1] * v[1]) + (v[2] * v[2] + v[3] * v[3])
                tot = lax.reduce_sum(s, (0,)) + jnp.zeros((16,), jnp.float32)
                tot2 = lax.reduce_sum(sq, (0,)) + jnp.zeros((16,), jnp.float32)
                mean = tot * inv_d
                var = tot2 * inv_d - mean * mean
                rstd = _rsqrt(jnp.maximum(var, 0.0) + EPS)
                for q in range(nq):
                    rows_v[r, pl.ds(q * 16, 16)] = (
                        (v[q] - mean) * (rstd * gq[q]) + bq[q]
                    )

            def row_body(g, carry2):
                for i in range(8):
                    one_row(g * 8 + i)
                return carry2

            lax.fori_loop(0, CHUNK // 8, row_body, 0)

        npairs = chunks // 2
        stage_and_fire(c0, 0, rows_a, gsem_a)

        def pair_body(t, carry):
            ca = c0 + 2 * t
            drain_gathers(rows_a, gsem_a)

            @pl.when(t > 0)
            def _():
                drain_write(rows_b, wsem_b)

            stage_and_fire(ca + 1, 1, rows_b, gsem_b)
            compute(rows_a)
            fire_write(ca, rows_a, wsem_a)
            drain_gathers(rows_b, gsem_b)

            @pl.when(t < npairs - 1)
            def _():
                drain_write(rows_a, wsem_a)
                stage_and_fire(ca + 2, 0, rows_a, gsem_a)

            compute(rows_b)
            fire_write(ca + 1, rows_b, wsem_b)
            return carry

        lax.fori_loop(0, npairs, pair_body, 0)
        drain_write(rows_a, wsem_a)
        drain_write(rows_b, wsem_b)

    return kern


def kernel(x, table, gamma, beta):
    b, s = x.shape
    num_emb, d = table.shape
    n_idx = b * s
    kern = _make_kernel(n_idx, num_emb, d)
    x_flat = x.reshape(n_idx // SUB, SUB)
    out = kern(x_flat, table, gamma, beta)
    return out.reshape(b, s, d)
